# Initial kernel scaffold; baseline (speedup 1.0000x reference)
import jax
import jax.numpy as jnp
from jax.experimental import pallas as pl


def kernel(hidden_states, q_lora, cos, sin, wq_b, wk, k_norm_w, k_norm_b, weights_proj):
    raise NotImplementedError("write your pallas kernel here")


# speedup vs baseline: 1.0261x; 1.0261x over previous
import functools
import numpy as np
import jax
import jax.numpy as jnp
from jax import lax
from jax.experimental import pallas as pl
from jax.experimental.pallas import tpu as pltpu

S = 2048
H = 2048
QLR = 1536
N = 16
D = 128
RD = 64
TOPK = 512

BQ = 256
BP = 512


def _ktr_body(k_ref, cos_ref, sin_ref, kt_ref):
    k = k_ref[...]
    c = cos_ref[...]
    s = sin_ref[...]
    k1 = k[:, :RD // 2]
    k2 = k[:, RD // 2:RD]
    kp = jnp.concatenate([k1 * c - k2 * s, k1 * s + k2 * c, k[:, RD:]], axis=1)
    kt_ref[...] = kp.T.astype(jnp.bfloat16)


def _wdot_body(h_ref, wp_ref, w_ref):
    w = jnp.dot(h_ref[...].astype(jnp.bfloat16),
                wp_ref[...].astype(jnp.bfloat16),
                preferred_element_type=jnp.float32)
    w_ref[...] = w * (N ** -0.5) * (D ** -0.5)


def _qdot_body(ql_ref, wqb_ref, q_ref):
    q_ref[...] = jnp.dot(ql_ref[...].astype(jnp.bfloat16),
                         wqb_ref[...].astype(jnp.bfloat16),
                         preferred_element_type=jnp.float32)


def _main_body(q_ref, cos_ref, sin_ref, kt_ref, w_ref, out_ref, key_ref):
    b = pl.program_id(0)
    q3 = q_ref[...].reshape(BQ, N, D)
    c = cos_ref[...][:, None, :]
    s = sin_ref[...][:, None, :]
    q1 = q3[:, :, :RD // 2]
    q2 = q3[:, :, RD // 2:RD]
    q3 = jnp.concatenate([q1 * c - q2 * s, q1 * s + q2 * c, q3[:, :, RD:]],
                         axis=2).astype(jnp.bfloat16)
    w = w_ref[...].astype(jnp.bfloat16).astype(jnp.float32)
    kt = kt_ref[...]
    terms = []
    for n in range(N):
        qn = q3[:, n, :]
        sc = jnp.dot(qn, kt, preferred_element_type=jnp.float32)
        sc = jnp.maximum(sc, 0.0).astype(jnp.bfloat16).astype(jnp.float32)
        terms.append(w[:, n:n + 1] * sc)
    stride = N // 2
    while stride:
        terms = [terms[i] + terms[i + stride] for i in range(stride)]
        stride //= 2
    acc = terms[0]
    row = b * BQ + lax.broadcasted_iota(jnp.int32, (BQ, S), 0)
    col = lax.broadcasted_iota(jnp.int32, (BQ, S), 1)
    acc = jnp.where(col > row, jnp.float32(-1e30), acc)
    out_ref[...] = acc
    u = lax.bitcast_convert_type(acc, jnp.int32)
    key_ref[...] = u ^ ((u >> 31) & jnp.int32(0x7FFFFFFF))


def _dense(hidden, q_lora, cos, sin, wq_b, wk, knw, knb, wproj):
    kraw = hidden @ wk
    mu = jnp.mean(kraw, axis=-1, keepdims=True)
    var = jnp.var(kraw, axis=-1, keepdims=True)
    k_ln = (kraw - mu) / jnp.sqrt(var + 1e-6) * knw + knb

    kt = pl.pallas_call(
        _ktr_body,
        grid=(S // BP,),
        in_specs=[
            pl.BlockSpec((BP, D), lambda j: (j, 0)),
            pl.BlockSpec((BP, RD // 2), lambda j: (j, 0)),
            pl.BlockSpec((BP, RD // 2), lambda j: (j, 0)),
        ],
        out_specs=pl.BlockSpec((D, BP), lambda j: (0, j)),
        out_shape=jax.ShapeDtypeStruct((D, S), jnp.bfloat16),
        compiler_params=pltpu.CompilerParams(
            dimension_semantics=("arbitrary",)),
    )(k_ln, cos, sin)

    w = pl.pallas_call(
        _wdot_body,
        grid=(S // BP,),
        in_specs=[
            pl.BlockSpec((BP, H), lambda j: (j, 0)),
            pl.BlockSpec((H, N), lambda j: (0, 0)),
        ],
        out_specs=pl.BlockSpec((BP, N), lambda j: (j, 0)),
        out_shape=jax.ShapeDtypeStruct((S, N), jnp.float32),
        compiler_params=pltpu.CompilerParams(
            dimension_semantics=("arbitrary",)),
    )(hidden, wproj)

    q_raw = pl.pallas_call(
        _qdot_body,
        grid=(S // BQ,),
        in_specs=[
            pl.BlockSpec((BQ, QLR), lambda b: (b, 0)),
            pl.BlockSpec((QLR, N * D), lambda b: (0, 0)),
        ],
        out_specs=pl.BlockSpec((BQ, N * D), lambda b: (b, 0)),
        out_shape=jax.ShapeDtypeStruct((S, N * D), jnp.float32),
        compiler_params=pltpu.CompilerParams(
            dimension_semantics=("arbitrary",)),
    )(q_lora, wq_b)

    score, key = pl.pallas_call(
        _main_body,
        grid=(S // BQ,),
        in_specs=[
            pl.BlockSpec((BQ, N * D), lambda b: (b, 0)),
            pl.BlockSpec((BQ, RD // 2), lambda b: (b, 0)),
            pl.BlockSpec((BQ, RD // 2), lambda b: (b, 0)),
            pl.BlockSpec((D, S), lambda b: (0, 0)),
            pl.BlockSpec((BQ, N), lambda b: (b, 0)),
        ],
        out_specs=[
            pl.BlockSpec((BQ, S), lambda b: (b, 0)),
            pl.BlockSpec((BQ, S), lambda b: (b, 0)),
        ],
        out_shape=[
            jax.ShapeDtypeStruct((S, S), jnp.float32),
            jax.ShapeDtypeStruct((S, S), jnp.int32),
        ],
        compiler_params=pltpu.CompilerParams(
            dimension_semantics=("arbitrary",)),
    )(q_raw, cos, sin, kt, w)
    return score, key


def kernel(hidden_states, q_lora, cos, sin, wq_b, wk, k_norm_w, k_norm_b,
           weights_proj):
    score, key = _dense(hidden_states[0], q_lora[0], cos, sin, wq_b, wk,
                        k_norm_w, k_norm_b, weights_proj)
    _, idx = jax.lax.top_k(score, TOPK)
    return score[None], idx[None]
